# baseline (speedup 1.0000x reference)
import functools

import jax
import jax.numpy as jnp
from jax import lax
from jax.experimental import pallas as pl
from jax.experimental.pallas import tpu as pltpu
from jax.experimental.pallas import tpu_sc as plsc

N_ENTS = 9000
N = 10000
H = 128
R = 16
B = 8
S = 16
E = 320000
CH = 128
NW = 32
NCHUNK = 80
E_PAD = NW * NCHUNK * CH
NBUF = 3
N_ACC = 10112
ROWS_PER_SUB = N_ACC // 16
NEG_SLOPE = (1.0 / 8.0 + 1.0 / 3.0) / 2.0


NROW = E_PAD // CH
IROW = NROW // R


def _prep_body(h_ref, w_ref, et_ref, src_ref, o_ref, g_ref, bd_ref):
    bd_ref[...] = jnp.zeros((H, H), jnp.float32)
    for b in range(B):
        bd_ref[b * S:(b + 1) * S, b * S:(b + 1) * S] = w_ref[0, b]
    o_ref[0] = jnp.dot(h_ref[...], bd_ref[...],
                       preferred_element_type=jnp.float32)
    g_ref[...] = et_ref[...] * N + src_ref[...]


def _build_table_gidx(h, w4, et2d, src2d):
    table, gidx2d = pl.pallas_call(
        _prep_body,
        grid=(R,),
        in_specs=[
            pl.BlockSpec((N, H), lambda r: (0, 0)),
            pl.BlockSpec((1, B, S, S), lambda r: (r, 0, 0, 0)),
            pl.BlockSpec((IROW, CH), lambda r: (r, 0)),
            pl.BlockSpec((IROW, CH), lambda r: (r, 0)),
        ],
        out_specs=[
            pl.BlockSpec((1, N, H), lambda r: (r, 0, 0)),
            pl.BlockSpec((IROW, CH), lambda r: (r, 0)),
        ],
        out_shape=[
            jax.ShapeDtypeStruct((R, N, H), jnp.float32),
            jax.ShapeDtypeStruct((NROW, CH), jnp.int32),
        ],
        scratch_shapes=[pltpu.VMEM((H, H), jnp.float32)],
    )(h, w4, et2d, src2d)
    return table.reshape(R * N, H), gidx2d


_mesh = plsc.VectorSubcoreMesh(core_axis_name="c", subcore_axis_name="s")


@functools.partial(
    pl.kernel,
    mesh=_mesh,
    out_type=jax.ShapeDtypeStruct((2 * N_ACC, H), jnp.float32),
    scratch_types=[
        pltpu.VMEM((NBUF, CH), jnp.int32),
        pltpu.VMEM((NBUF, CH), jnp.int32),
        pltpu.VMEM((NBUF, CH, H), jnp.float32),
        pltpu.VMEM_SHARED((N_ACC, H), jnp.float32),
    ]
    + [pltpu.SemaphoreType.DMA] * (3 * NBUF),
)
def _sc_gather_scatter(table, gidx2d, dst2d, zrows, out, gbuf, dbuf, rows,
                       acc, *sems):
    gsem = sems[:NBUF]
    igsem = sems[NBUF:2 * NBUF]
    idsem = sems[2 * NBUF:]
    c = lax.axis_index("c")
    s = lax.axis_index("s")
    wid = s * 2 + c
    pltpu.sync_copy(zrows, acc.at[pl.ds(s * ROWS_PER_SUB, ROWS_PER_SUB)])
    plsc.subcore_barrier()

    base = wid * NCHUNK
    for q in range(NBUF):
        pltpu.sync_copy(gidx2d.at[base + q], gbuf.at[q])
        pltpu.sync_copy(dst2d.at[base + q], dbuf.at[q])
        pltpu.async_copy(table.at[gbuf.at[q]], rows.at[q], gsem[q])

    ngrp = (NCHUNK + NBUF - 1) // NBUF

    def body(p, carry):
        for q in range(NBUF):
            j = p * NBUF + q

            @pl.when(j < NCHUNK)
            def _():
                pltpu.make_async_copy(table.at[gbuf.at[q]], rows.at[q],
                                      gsem[q]).wait()
                more = j + NBUF < NCHUNK

                @pl.when(more)
                def _():
                    pltpu.async_copy(gidx2d.at[base + j + NBUF],
                                     gbuf.at[q], igsem[q])
                pltpu.sync_copy(rows.at[q], acc.at[dbuf.at[q]], add=True)

                @pl.when(more)
                def _():
                    pltpu.async_copy(dst2d.at[base + j + NBUF],
                                     dbuf.at[q], idsem[q])
                    pltpu.make_async_copy(gidx2d.at[base + j + NBUF],
                                          gbuf.at[q], igsem[q]).wait()
                    pltpu.async_copy(table.at[gbuf.at[q]], rows.at[q],
                                     gsem[q])
                    pltpu.make_async_copy(dst2d.at[base + j + NBUF],
                                          dbuf.at[q], idsem[q]).wait()
        return carry

    lax.fori_loop(0, ngrp, body, 0)
    plsc.subcore_barrier()
    base = c * N_ACC + s * ROWS_PER_SUB
    pltpu.sync_copy(acc.at[pl.ds(s * ROWS_PER_SUB, ROWS_PER_SUB)],
                    out.at[pl.ds(base, ROWS_PER_SUB)])


CHF = 1000


def _final_body(p_ref, n_ref, o_ref):
    x = p_ref[0] + p_ref[1]
    x = x * n_ref[...]
    x = jnp.where(x >= 0, x, x * NEG_SLOPE)
    ss = jnp.sum(x * x, axis=1, keepdims=True)
    o_ref[...] = x / jnp.maximum(jnp.sqrt(ss), 1e-12)


def _finish(partials, norm):
    return pl.pallas_call(
        _final_body,
        grid=(N_ENTS // CHF,),
        in_specs=[
            pl.BlockSpec((2, CHF, H), lambda i: (0, i, 0)),
            pl.BlockSpec((CHF, 1), lambda i: (i, 0)),
        ],
        out_specs=pl.BlockSpec((CHF, H), lambda i: (i, 0)),
        out_shape=jax.ShapeDtypeStruct((N_ENTS, H), jnp.float32),
    )(partials, norm)


def kernel(edge_index, edge_type, norm, dynamic_emb, words_emb, rel_weight):
    h = jnp.concatenate([dynamic_emb, words_emb], axis=0)
    src = edge_index[0].astype(jnp.int32)
    dst = edge_index[1].astype(jnp.int32)
    et = edge_type.astype(jnp.int32)

    pad = E_PAD - E
    ar = jnp.arange(pad, dtype=jnp.int32)
    src_p = jnp.concatenate([src, ar % N])
    et_p = jnp.concatenate([et, jnp.zeros((pad,), jnp.int32)])
    dst_p = jnp.concatenate([dst, N + ar % (N_ACC - N)])

    w4 = rel_weight.reshape(R, B, S, S)
    table, gidx2d = _build_table_gidx(
        h, w4, et_p.reshape(-1, CH), src_p.reshape(-1, CH))
    dst2d = dst_p.reshape(-1, CH)

    zrows = jnp.zeros((ROWS_PER_SUB, H), jnp.float32)
    flat = _sc_gather_scatter(table, gidx2d, dst2d, zrows)
    partials = flat.reshape(2, N_ACC, H)

    static_emb = _finish(partials, norm[:N_ENTS])
    return (static_emb, static_emb)


# speedup vs baseline: 1.0082x; 1.0082x over previous
import functools

import jax
import jax.numpy as jnp
from jax import lax
from jax.experimental import pallas as pl
from jax.experimental.pallas import tpu as pltpu
from jax.experimental.pallas import tpu_sc as plsc

N_ENTS = 9000
N = 10000
H = 128
R = 16
B = 8
S = 16
E = 320000
CH = 128
NW = 32
NCHUNK = 80
E_PAD = NW * NCHUNK * CH
NBUF = 3
N_ACC = 10112
ROWS_PER_SUB = N_ACC // 16
NEG_SLOPE = (1.0 / 8.0 + 1.0 / 3.0) / 2.0


NROW = E_PAD // CH
IROW = NROW // R


def _prep_body(h_ref, w_ref, et_ref, src_ref, dst_ref, o_ref, g_ref, d_ref,
               bd_ref):
    bd_ref[...] = jnp.zeros((H, H), jnp.float32)
    for b in range(B):
        bd_ref[b * S:(b + 1) * S, b * S:(b + 1) * S] = w_ref[0, b]
    o_ref[0] = jnp.dot(h_ref[...], bd_ref[...],
                       preferred_element_type=jnp.float32)
    r = pl.program_id(0)
    row = jax.lax.broadcasted_iota(jnp.int32, (IROW, CH), 0) + r * IROW
    eid = row * CH + jax.lax.broadcasted_iota(jnp.int32, (IROW, CH), 1)
    valid = eid < E
    padi = jnp.maximum(eid - E, 0)
    g_ref[...] = jnp.where(valid, et_ref[...] * N + src_ref[...], padi % N)
    d_ref[...] = jnp.where(valid, dst_ref[...], N + padi % (N_ACC - N))


def _build_table_gidx(h, w4, et2d, src2d, dst2d):
    nreal = E // CH
    return pl.pallas_call(
        _prep_body,
        grid=(R,),
        in_specs=[
            pl.BlockSpec((N, H), lambda r: (0, 0)),
            pl.BlockSpec((1, B, S, S), lambda r: (r, 0, 0, 0)),
            pl.BlockSpec((IROW, CH), lambda r: (r, 0)),
            pl.BlockSpec((IROW, CH), lambda r: (r, 0)),
            pl.BlockSpec((IROW, CH), lambda r: (r, 0)),
        ],
        out_specs=[
            pl.BlockSpec((1, N, H), lambda r: (r, 0, 0)),
            pl.BlockSpec((IROW, CH), lambda r: (r, 0)),
            pl.BlockSpec((IROW, CH), lambda r: (r, 0)),
        ],
        out_shape=[
            jax.ShapeDtypeStruct((R, N, H), jnp.float32),
            jax.ShapeDtypeStruct((NROW, CH), jnp.int32),
            jax.ShapeDtypeStruct((NROW, CH), jnp.int32),
        ],
        scratch_shapes=[pltpu.VMEM((H, H), jnp.float32)],
    )(h, w4, et2d, src2d, dst2d)


_mesh = plsc.VectorSubcoreMesh(core_axis_name="c", subcore_axis_name="s")


@functools.partial(
    pl.kernel,
    mesh=_mesh,
    out_type=jax.ShapeDtypeStruct((2 * N_ACC, H), jnp.float32),
    scratch_types=[
        pltpu.VMEM((NBUF, CH), jnp.int32),
        pltpu.VMEM((NBUF, CH), jnp.int32),
        pltpu.VMEM((NBUF, CH, H), jnp.float32),
        pltpu.VMEM_SHARED((N_ACC, H), jnp.float32),
    ]
    + [pltpu.SemaphoreType.DMA] * (3 * NBUF),
)
def _sc_gather_scatter(table, gidx2d, dst2d, zrows, out, gbuf, dbuf, rows,
                       acc, *sems):
    gsem = sems[:NBUF]
    igsem = sems[NBUF:2 * NBUF]
    idsem = sems[2 * NBUF:]
    c = lax.axis_index("c")
    s = lax.axis_index("s")
    wid = s * 2 + c
    pltpu.sync_copy(zrows, acc.at[pl.ds(s * ROWS_PER_SUB, ROWS_PER_SUB)])
    plsc.subcore_barrier()

    base = wid * NCHUNK
    for q in range(NBUF):
        pltpu.sync_copy(gidx2d.at[base + q], gbuf.at[q])
        pltpu.sync_copy(dst2d.at[base + q], dbuf.at[q])
        pltpu.async_copy(table.at[gbuf.at[q]], rows.at[q], gsem[q])

    ngrp = (NCHUNK + NBUF - 1) // NBUF

    def body(p, carry):
        for q in range(NBUF):
            j = p * NBUF + q

            @pl.when(j < NCHUNK)
            def _():
                pltpu.make_async_copy(table.at[gbuf.at[q]], rows.at[q],
                                      gsem[q]).wait()
                more = j + NBUF < NCHUNK

                @pl.when(more)
                def _():
                    pltpu.async_copy(gidx2d.at[base + j + NBUF],
                                     gbuf.at[q], igsem[q])
                pltpu.sync_copy(rows.at[q], acc.at[dbuf.at[q]], add=True)

                @pl.when(more)
                def _():
                    pltpu.async_copy(dst2d.at[base + j + NBUF],
                                     dbuf.at[q], idsem[q])
                    pltpu.make_async_copy(gidx2d.at[base + j + NBUF],
                                          gbuf.at[q], igsem[q]).wait()
                    pltpu.async_copy(table.at[gbuf.at[q]], rows.at[q],
                                     gsem[q])
                    pltpu.make_async_copy(dst2d.at[base + j + NBUF],
                                          dbuf.at[q], idsem[q]).wait()
        return carry

    lax.fori_loop(0, ngrp, body, 0)
    plsc.subcore_barrier()
    base = c * N_ACC + s * ROWS_PER_SUB
    pltpu.sync_copy(acc.at[pl.ds(s * ROWS_PER_SUB, ROWS_PER_SUB)],
                    out.at[pl.ds(base, ROWS_PER_SUB)])


CHF = 1000


def _final_body(p_ref, n_ref, o_ref):
    x = p_ref[0] + p_ref[1]
    x = x * n_ref[...]
    x = jnp.where(x >= 0, x, x * NEG_SLOPE)
    ss = jnp.sum(x * x, axis=1, keepdims=True)
    o_ref[...] = x / jnp.maximum(jnp.sqrt(ss), 1e-12)


def _finish(partials, norm):
    return pl.pallas_call(
        _final_body,
        grid=(N_ENTS // CHF,),
        in_specs=[
            pl.BlockSpec((2, CHF, H), lambda i: (0, i, 0)),
            pl.BlockSpec((CHF, 1), lambda i: (i, 0)),
        ],
        out_specs=pl.BlockSpec((CHF, H), lambda i: (i, 0)),
        out_shape=jax.ShapeDtypeStruct((N_ENTS, H), jnp.float32),
    )(partials, norm)


def kernel(edge_index, edge_type, norm, dynamic_emb, words_emb, rel_weight):
    h = jnp.concatenate([dynamic_emb, words_emb], axis=0)
    src2d = edge_index[0].astype(jnp.int32).reshape(-1, CH)
    dst2d_raw = edge_index[1].astype(jnp.int32).reshape(-1, CH)
    et2d = edge_type.astype(jnp.int32).reshape(-1, CH)

    w4 = rel_weight.reshape(R, B, S, S)
    table4, gidx2d, dst2d = _build_table_gidx(h, w4, et2d, src2d, dst2d_raw)
    table = table4.reshape(R * N, H)

    zrows = jnp.zeros((ROWS_PER_SUB, H), jnp.float32)
    flat = _sc_gather_scatter(table, gidx2d, dst2d, zrows)
    partials = flat.reshape(2, N_ACC, H)

    static_emb = _finish(partials, norm[:N_ENTS])
    return (static_emb, static_emb)


# speedup vs baseline: 1.0143x; 1.0060x over previous
import functools

import jax
import jax.numpy as jnp
from jax import lax
from jax.experimental import pallas as pl
from jax.experimental.pallas import tpu as pltpu
from jax.experimental.pallas import tpu_sc as plsc

N_ENTS = 9000
N = 10000
H = 128
R = 16
B = 8
S = 16
E = 320000
CH = 128
NW = 32
NCHUNK = 80
E_PAD = NW * NCHUNK * CH
NBUF = 3
N_ACC = 10112
ROWS_PER_SUB = N_ACC // 16
NEG_SLOPE = (1.0 / 8.0 + 1.0 / 3.0) / 2.0


NROW = E_PAD // CH
IROW = NROW // R


def _prep_body(de_ref, we_ref, w_ref, et_ref, src_ref, dst_ref, o_ref,
               g_ref, d_ref, bd_ref):
    bd_ref[...] = jnp.zeros((H, H), jnp.float32)
    for b in range(B):
        bd_ref[b * S:(b + 1) * S, b * S:(b + 1) * S] = w_ref[0, b]
    bd = bd_ref[...]
    o_ref[0, :N_ENTS] = jnp.dot(de_ref[...], bd,
                                preferred_element_type=jnp.float32)
    o_ref[0, N_ENTS:] = jnp.dot(we_ref[...], bd,
                                preferred_element_type=jnp.float32)
    r = pl.program_id(0)
    row = jax.lax.broadcasted_iota(jnp.int32, (IROW, CH), 0) + r * IROW
    eid = row * CH + jax.lax.broadcasted_iota(jnp.int32, (IROW, CH), 1)
    valid = eid < E
    padi = jnp.maximum(eid - E, 0)
    g_ref[...] = jnp.where(valid, et_ref[...] * N + src_ref[...], padi % N)
    d_ref[...] = jnp.where(valid, dst_ref[...], N + padi % (N_ACC - N))


def _build_table_gidx(de, we, w4, et2d, src2d, dst2d):
    return pl.pallas_call(
        _prep_body,
        grid=(R,),
        in_specs=[
            pl.BlockSpec((N_ENTS, H), lambda r: (0, 0)),
            pl.BlockSpec((N - N_ENTS, H), lambda r: (0, 0)),
            pl.BlockSpec((1, B, S, S), lambda r: (r, 0, 0, 0)),
            pl.BlockSpec((IROW, CH), lambda r: (r, 0)),
            pl.BlockSpec((IROW, CH), lambda r: (r, 0)),
            pl.BlockSpec((IROW, CH), lambda r: (r, 0)),
        ],
        out_specs=[
            pl.BlockSpec((1, N, H), lambda r: (r, 0, 0)),
            pl.BlockSpec((IROW, CH), lambda r: (r, 0)),
            pl.BlockSpec((IROW, CH), lambda r: (r, 0)),
        ],
        out_shape=[
            jax.ShapeDtypeStruct((R, N, H), jnp.float32),
            jax.ShapeDtypeStruct((NROW, CH), jnp.int32),
            jax.ShapeDtypeStruct((NROW, CH), jnp.int32),
        ],
        scratch_shapes=[pltpu.VMEM((H, H), jnp.float32)],
    )(de, we, w4, et2d, src2d, dst2d)


_mesh = plsc.VectorSubcoreMesh(core_axis_name="c", subcore_axis_name="s")


@functools.partial(
    pl.kernel,
    mesh=_mesh,
    out_type=jax.ShapeDtypeStruct((2 * N_ACC, H), jnp.float32),
    scratch_types=[
        pltpu.VMEM((NBUF, CH), jnp.int32),
        pltpu.VMEM((NBUF, CH), jnp.int32),
        pltpu.VMEM((NBUF, CH, H), jnp.float32),
        pltpu.VMEM_SHARED((N_ACC, H), jnp.float32),
    ]
    + [pltpu.SemaphoreType.DMA] * (3 * NBUF),
)
def _sc_gather_scatter(table, gidx2d, dst2d, zrows, out, gbuf, dbuf, rows,
                       acc, *sems):
    gsem = sems[:NBUF]
    igsem = sems[NBUF:2 * NBUF]
    idsem = sems[2 * NBUF:]
    c = lax.axis_index("c")
    s = lax.axis_index("s")
    wid = s * 2 + c
    pltpu.sync_copy(zrows, acc.at[pl.ds(s * ROWS_PER_SUB, ROWS_PER_SUB)])
    plsc.subcore_barrier()

    base = wid * NCHUNK
    for q in range(NBUF):
        pltpu.sync_copy(gidx2d.at[base + q], gbuf.at[q])
        pltpu.sync_copy(dst2d.at[base + q], dbuf.at[q])
        pltpu.async_copy(table.at[gbuf.at[q]], rows.at[q], gsem[q])

    ngrp = (NCHUNK + NBUF - 1) // NBUF

    def body(p, carry):
        for q in range(NBUF):
            j = p * NBUF + q

            @pl.when(j < NCHUNK)
            def _():
                pltpu.make_async_copy(table.at[gbuf.at[q]], rows.at[q],
                                      gsem[q]).wait()
                more = j + NBUF < NCHUNK

                @pl.when(more)
                def _():
                    pltpu.async_copy(gidx2d.at[base + j + NBUF],
                                     gbuf.at[q], igsem[q])
                pltpu.sync_copy(rows.at[q], acc.at[dbuf.at[q]], add=True)

                @pl.when(more)
                def _():
                    pltpu.async_copy(dst2d.at[base + j + NBUF],
                                     dbuf.at[q], idsem[q])
                    pltpu.make_async_copy(gidx2d.at[base + j + NBUF],
                                          gbuf.at[q], igsem[q]).wait()
                    pltpu.async_copy(table.at[gbuf.at[q]], rows.at[q],
                                     gsem[q])
                    pltpu.make_async_copy(dst2d.at[base + j + NBUF],
                                          dbuf.at[q], idsem[q]).wait()
        return carry

    lax.fori_loop(0, ngrp, body, 0)
    plsc.subcore_barrier()
    base = c * N_ACC + s * ROWS_PER_SUB
    pltpu.sync_copy(acc.at[pl.ds(s * ROWS_PER_SUB, ROWS_PER_SUB)],
                    out.at[pl.ds(base, ROWS_PER_SUB)])


CHF = 1000


def _final_body(p_ref, n_ref, o_ref):
    x = p_ref[0] + p_ref[1]
    x = x * n_ref[...]
    x = jnp.where(x >= 0, x, x * NEG_SLOPE)
    ss = jnp.sum(x * x, axis=1, keepdims=True)
    o_ref[...] = x / jnp.maximum(jnp.sqrt(ss), 1e-12)


def _finish(partials, norm):
    return pl.pallas_call(
        _final_body,
        grid=(N_ENTS // CHF,),
        in_specs=[
            pl.BlockSpec((2, CHF, H), lambda i: (0, i, 0)),
            pl.BlockSpec((CHF, 1), lambda i: (i, 0)),
        ],
        out_specs=pl.BlockSpec((CHF, H), lambda i: (i, 0)),
        out_shape=jax.ShapeDtypeStruct((N_ENTS, H), jnp.float32),
    )(partials, norm)


def kernel(edge_index, edge_type, norm, dynamic_emb, words_emb, rel_weight):
    src2d = edge_index[0].astype(jnp.int32).reshape(-1, CH)
    dst2d_raw = edge_index[1].astype(jnp.int32).reshape(-1, CH)
    et2d = edge_type.astype(jnp.int32).reshape(-1, CH)

    w4 = rel_weight.reshape(R, B, S, S)
    table4, gidx2d, dst2d = _build_table_gidx(
        dynamic_emb, words_emb, w4, et2d, src2d, dst2d_raw)
    table = table4.reshape(R * N, H)

    zrows = jnp.zeros((ROWS_PER_SUB, H), jnp.float32)
    flat = _sc_gather_scatter(table, gidx2d, dst2d, zrows)
    partials = flat.reshape(2, N_ACC, H)

    static_emb = _finish(partials, norm[:N_ENTS])
    return (static_emb, static_emb)


# speedup vs baseline: 1.0250x; 1.0106x over previous
import functools

import jax
import jax.numpy as jnp
from jax import lax
from jax.experimental import pallas as pl
from jax.experimental.pallas import tpu as pltpu
from jax.experimental.pallas import tpu_sc as plsc

N_ENTS = 9000
N = 10000
H = 128
R = 16
B = 8
S = 16
E = 320000
CH = 128
NW = 32
NCHUNK = 80
E_PAD = NW * NCHUNK * CH
NBUF = 3
N_ACC = 10112
ROWS_PER_SUB = N_ACC // 16
NEG_SLOPE = (1.0 / 8.0 + 1.0 / 3.0) / 2.0


NROW = E_PAD // CH
IROW = NROW // R


def _prep_body(de_ref, we_ref, w_ref, et_ref, src_ref, dst_ref, o_ref,
               g_ref, d_ref, bd_ref):
    bd_ref[...] = jnp.zeros((H, H), jnp.float32)
    for b in range(B):
        bd_ref[b * S:(b + 1) * S, b * S:(b + 1) * S] = w_ref[0, b]
    bd = bd_ref[...]
    o_ref[0, :N_ENTS] = jnp.dot(de_ref[...], bd,
                                preferred_element_type=jnp.float32)
    o_ref[0, N_ENTS:] = jnp.dot(we_ref[...], bd,
                                preferred_element_type=jnp.float32)
    r = pl.program_id(0)
    row = jax.lax.broadcasted_iota(jnp.int32, (IROW, CH), 0) + r * IROW
    eid = row * CH + jax.lax.broadcasted_iota(jnp.int32, (IROW, CH), 1)
    valid = eid < E
    padi = jnp.maximum(eid - E, 0)
    g_ref[...] = jnp.where(valid, et_ref[...] * N + src_ref[...], padi % N)
    d_ref[...] = jnp.where(valid, dst_ref[...], N + padi % (N_ACC - N))


def _build_table_gidx(de, we, w4, et2d, src2d, dst2d):
    return pl.pallas_call(
        _prep_body,
        grid=(R,),
        in_specs=[
            pl.BlockSpec((N_ENTS, H), lambda r: (0, 0)),
            pl.BlockSpec((N - N_ENTS, H), lambda r: (0, 0)),
            pl.BlockSpec((1, B, S, S), lambda r: (r, 0, 0, 0)),
            pl.BlockSpec((IROW, CH), lambda r: (r, 0)),
            pl.BlockSpec((IROW, CH), lambda r: (r, 0)),
            pl.BlockSpec((IROW, CH), lambda r: (r, 0)),
        ],
        out_specs=[
            pl.BlockSpec((1, N, H), lambda r: (r, 0, 0)),
            pl.BlockSpec((IROW, CH), lambda r: (r, 0)),
            pl.BlockSpec((IROW, CH), lambda r: (r, 0)),
        ],
        out_shape=[
            jax.ShapeDtypeStruct((R, N, H), jnp.float32),
            jax.ShapeDtypeStruct((NROW, CH), jnp.int32),
            jax.ShapeDtypeStruct((NROW, CH), jnp.int32),
        ],
        scratch_shapes=[pltpu.VMEM((H, H), jnp.float32)],
    )(de, we, w4, et2d, src2d, dst2d)


_mesh = plsc.VectorSubcoreMesh(core_axis_name="c", subcore_axis_name="s")


@functools.partial(
    pl.kernel,
    mesh=_mesh,
    out_type=jax.ShapeDtypeStruct((2 * N_ACC, H), jnp.float32),
    scratch_types=[
        pltpu.VMEM((NBUF, CH), jnp.int32),
        pltpu.VMEM((NBUF, CH), jnp.int32),
        pltpu.VMEM((NBUF, CH, H), jnp.float32),
        pltpu.VMEM_SHARED((N_ACC, H), jnp.float32),
    ]
    + [pltpu.SemaphoreType.DMA] * (3 * NBUF),
)
def _sc_gather_scatter(table, gidx2d, dst2d, zrows, out, gbuf, dbuf, rows,
                       acc, *sems):
    gsem = sems[:NBUF]
    igsem = sems[NBUF:2 * NBUF]
    idsem = sems[2 * NBUF:]
    c = lax.axis_index("c")
    s = lax.axis_index("s")
    wid = s * 2 + c
    pltpu.sync_copy(zrows, acc.at[pl.ds(s * ROWS_PER_SUB, ROWS_PER_SUB)])
    plsc.subcore_barrier()

    base = wid * NCHUNK
    for q in range(NBUF):
        pltpu.sync_copy(gidx2d.at[base + q], gbuf.at[q])
        pltpu.sync_copy(dst2d.at[base + q], dbuf.at[q])
        pltpu.async_copy(table.at[gbuf.at[q]], rows.at[q], gsem[q])

    ngrp = (NCHUNK + NBUF - 1) // NBUF

    def body(p, carry):
        for q in range(NBUF):
            j = p * NBUF + q

            @pl.when(j < NCHUNK)
            def _():
                pltpu.make_async_copy(table.at[gbuf.at[q]], rows.at[q],
                                      gsem[q]).wait()
                more = j + NBUF < NCHUNK

                @pl.when(more)
                def _():
                    pltpu.async_copy(gidx2d.at[base + j + NBUF],
                                     gbuf.at[q], igsem[q])
                pltpu.sync_copy(rows.at[q], acc.at[dbuf.at[q]], add=True)

                @pl.when(more)
                def _():
                    pltpu.async_copy(dst2d.at[base + j + NBUF],
                                     dbuf.at[q], idsem[q])
                    pltpu.make_async_copy(gidx2d.at[base + j + NBUF],
                                          gbuf.at[q], igsem[q]).wait()
                    pltpu.async_copy(table.at[gbuf.at[q]], rows.at[q],
                                     gsem[q])
                    pltpu.make_async_copy(dst2d.at[base + j + NBUF],
                                          dbuf.at[q], idsem[q]).wait()
        return carry

    lax.fori_loop(0, ngrp, body, 0)
    plsc.subcore_barrier()
    base = c * N_ACC + s * ROWS_PER_SUB
    pltpu.sync_copy(acc.at[pl.ds(s * ROWS_PER_SUB, ROWS_PER_SUB)],
                    out.at[pl.ds(base, ROWS_PER_SUB)])


CHF = 3000


def _final_body(p_ref, n_ref, o_ref):
    x = p_ref[0] + p_ref[1]
    x = x * n_ref[...]
    x = jnp.where(x >= 0, x, x * NEG_SLOPE)
    ss = jnp.sum(x * x, axis=1, keepdims=True)
    o_ref[...] = x / jnp.maximum(jnp.sqrt(ss), 1e-12)


def _finish(partials, norm):
    return pl.pallas_call(
        _final_body,
        grid=(N_ENTS // CHF,),
        in_specs=[
            pl.BlockSpec((2, CHF, H), lambda i: (0, i, 0)),
            pl.BlockSpec((CHF, 1), lambda i: (i, 0)),
        ],
        out_specs=pl.BlockSpec((CHF, H), lambda i: (i, 0)),
        out_shape=jax.ShapeDtypeStruct((N_ENTS, H), jnp.float32),
    )(partials, norm)


def kernel(edge_index, edge_type, norm, dynamic_emb, words_emb, rel_weight):
    src2d = edge_index[0].astype(jnp.int32).reshape(-1, CH)
    dst2d_raw = edge_index[1].astype(jnp.int32).reshape(-1, CH)
    et2d = edge_type.astype(jnp.int32).reshape(-1, CH)

    w4 = rel_weight.reshape(R, B, S, S)
    table4, gidx2d, dst2d = _build_table_gidx(
        dynamic_emb, words_emb, w4, et2d, src2d, dst2d_raw)
    table = table4.reshape(R * N, H)

    zrows = jnp.zeros((ROWS_PER_SUB, H), jnp.float32)
    flat = _sc_gather_scatter(table, gidx2d, dst2d, zrows)
    partials = flat.reshape(2, N_ACC, H)

    static_emb = _finish(partials, norm[:N_ENTS])
    return (static_emb, static_emb)
